# Initial kernel scaffold; baseline (speedup 1.0000x reference)
import jax
import jax.numpy as jnp
from jax.experimental import pallas as pl


def kernel(x, table):
    raise NotImplementedError("write your pallas kernel here")


# speedup vs baseline: 1.7699x; 1.7699x over previous
import functools

import jax
import jax.numpy as jnp
from jax import lax
from jax.experimental import pallas as pl
from jax.experimental.pallas import tpu as pltpu
from jax.experimental.pallas import tpu_sc as plsc

DIM = 2048
NC = 2
NS = 16
NW = NC * NS

CHUNK = 16


@functools.lru_cache(maxsize=None)
def _make_gather(B):
    b_per_w = B // NW
    n_chunks = b_per_w // CHUNK
    mesh = plsc.VectorSubcoreMesh(core_axis_name="c", subcore_axis_name="s")

    @functools.partial(
        pl.kernel,
        mesh=mesh,
        out_type=jax.ShapeDtypeStruct((B, DIM), jnp.float32),
        scratch_types=[
            pltpu.VMEM((b_per_w,), jnp.int32),
            pltpu.VMEM((2, CHUNK, DIM), jnp.float32),
            pltpu.SemaphoreType.DMA,
            pltpu.SemaphoreType.DMA,
        ],
    )
    def gather_kernel(idx_hbm, table_hbm, out_hbm, idx_v, buf_v, gs0, gs1):
        wid = lax.axis_index("s") * NC + lax.axis_index("c")
        base = wid * b_per_w
        pltpu.sync_copy(idx_hbm.at[pl.ds(base, b_per_w)], idx_v)

        def gather_desc(g, b, sem):
            return pltpu.make_async_copy(
                table_hbm.at[idx_v.at[pl.ds(g * CHUNK, CHUNK)]],
                buf_v.at[b],
                sem,
            )

        gather_desc(0, 0, gs0).start()
        gather_desc(1, 1, gs1).start()

        def outer(i, carry):
            for b in range(2):
                g = 2 * i + b
                sem = gs0 if b == 0 else gs1
                gather_desc(g, b, sem).wait()
                pltpu.sync_copy(
                    buf_v.at[b], out_hbm.at[pl.ds(base + g * CHUNK, CHUNK)]
                )

                @pl.when(g + 2 < n_chunks)
                def _(b=b, g=g, sem=sem):
                    gather_desc(g + 2, b, sem).start()

            return carry

        lax.fori_loop(0, n_chunks // 2, outer, 0)

    return gather_kernel


@jax.jit
def kernel(x, table):
    idx = x.reshape(-1).astype(jnp.int32)
    out = _make_gather(idx.shape[0])(idx, table)
    return out.reshape(*x.shape, DIM)


# speedup vs baseline: 1.7706x; 1.0004x over previous
import functools

import jax
import jax.numpy as jnp
from jax import lax
from jax.experimental import pallas as pl
from jax.experimental.pallas import tpu as pltpu
from jax.experimental.pallas import tpu_sc as plsc

DIM = 2048
NC = 2
NS = 16
NW = NC * NS

CHUNK = 8
NBUF = 4


@functools.lru_cache(maxsize=None)
def _make_gather(B):
    b_per_w = B // NW
    n_chunks = b_per_w // CHUNK
    assert n_chunks % NBUF == 0
    mesh = plsc.VectorSubcoreMesh(core_axis_name="c", subcore_axis_name="s")

    @functools.partial(
        pl.kernel,
        mesh=mesh,
        out_type=jax.ShapeDtypeStruct((B, DIM), jnp.float32),
        scratch_types=[
            pltpu.VMEM((b_per_w,), jnp.int32),
            pltpu.VMEM((NBUF, CHUNK, DIM), jnp.float32),
        ]
        + [pltpu.SemaphoreType.DMA] * (2 * NBUF),
    )
    def gather_kernel(idx_hbm, table_hbm, out_hbm, idx_v, buf_v, *sems):
        gsems, wsems = sems[:NBUF], sems[NBUF:]
        wid = lax.axis_index("s") * NC + lax.axis_index("c")
        base = wid * b_per_w
        pltpu.sync_copy(idx_hbm.at[pl.ds(base, b_per_w)], idx_v)

        def gather_desc(g, b):
            return pltpu.make_async_copy(
                table_hbm.at[idx_v.at[pl.ds(g * CHUNK, CHUNK)]],
                buf_v.at[b],
                gsems[b],
            )

        def write_desc(g, b):
            return pltpu.make_async_copy(
                buf_v.at[b],
                out_hbm.at[pl.ds(base + g * CHUNK, CHUNK)],
                wsems[b],
            )

        gather_desc(0, 0).start()
        gather_desc(1, 1).start()

        def outer(i, carry):
            for j in range(NBUF):
                g = NBUF * i + j
                gather_desc(g, j).wait()
                write_desc(g, j).start()

                bn = (j + 2) % NBUF

                @pl.when(g >= 2)
                def _(g=g, bn=bn):
                    write_desc(g - 2, bn).wait()

                @pl.when(g + 2 < n_chunks)
                def _(g=g, bn=bn):
                    gather_desc(g + 2, bn).start()

            return carry

        lax.fori_loop(0, n_chunks // NBUF, outer, 0)
        write_desc(n_chunks - 2, (n_chunks - 2) % NBUF).wait()
        write_desc(n_chunks - 1, (n_chunks - 1) % NBUF).wait()

    return gather_kernel


@jax.jit
def kernel(x, table):
    idx = x.reshape(-1).astype(jnp.int32)
    out = _make_gather(idx.shape[0])(idx, table)
    return out.reshape(*x.shape, DIM)
